# baseline (speedup 1.0000x reference)
import jax
import jax.numpy as jnp
from jax import lax
from jax.experimental import pallas as pl
from jax.experimental.pallas import tpu as pltpu
from jax.experimental.pallas import tpu_sc as plsc

F32 = jnp.float32
NC, NS = 2, 16
BATCH = 128
SLAB = 13


def _seg_sum(n_nodes, n_rows, h, with_ea):
    nw = NC * NS
    rows_per = n_rows // nw
    tail = n_rows - rows_per * nw
    n_slabs = rows_per // SLAB
    rem_rows = rows_per - n_slabs * SLAB
    per_sub = -(-n_nodes // NS)
    per_sub += (-per_sub) % 8
    n_pad = per_sub * NS

    mesh = plsc.VectorSubcoreMesh(core_axis_name="c", subcore_axis_name="s")
    out_types = [jax.ShapeDtypeStruct((NC, n_pad, h), F32)]
    scratch = [
        pltpu.VMEM((per_sub, h), F32),
        pltpu.VMEM((SLAB, 1, BATCH), jnp.int32),
        pltpu.VMEM((SLAB, 1, BATCH), jnp.int32),
        pltpu.VMEM((SLAB * BATCH, h), F32),
        pltpu.SemaphoreType.DMA,
        pltpu.VMEM_SHARED((n_pad, h), F32),
    ]
    if with_ea:
        out_types.append(jax.ShapeDtypeStruct((NC, n_pad, h), F32))
        scratch += [
            pltpu.VMEM((SLAB * BATCH, h), F32),
            pltpu.VMEM_SHARED((n_pad, h), F32),
        ]

    def body(*refs):
        if with_ea:
            (table, src1, dst1, ea, outg, oute,
             zbuf, didx, sidx, rows, sem, accg, earows, acce) = refs
        else:
            (table, src1, dst1, outg,
             zbuf, didx, sidx, rows, sem, accg) = refs
        c = lax.axis_index("c")
        s = lax.axis_index("s")
        wid = c * NS + s

        def zloop(i, carry):
            zbuf[i] = jnp.zeros((h,), F32)
            return carry
        lax.fori_loop(0, per_sub, zloop, 0)
        sl = pl.ds(s * per_sub, per_sub)
        pltpu.sync_copy(zbuf, accg.at[sl])
        if with_ea:
            pltpu.sync_copy(zbuf, acce.at[sl])
        plsc.subcore_barrier()

        def do_slab(r0, nr):
            for j in range(nr):
                pltpu.sync_copy(dst1.at[pl.ds((r0 + j) * BATCH, BATCH)],
                                didx.at[j, 0])
                pltpu.sync_copy(src1.at[pl.ds((r0 + j) * BATCH, BATCH)],
                                sidx.at[j, 0])
            if with_ea:
                pltpu.sync_copy(ea.at[pl.ds(r0 * BATCH, nr * BATCH)],
                                earows.at[pl.ds(0, nr * BATCH)])
            descs = [pltpu.async_copy(table.at[sidx.at[j, 0]],
                                      rows.at[pl.ds(j * BATCH, BATCH)], sem)
                     for j in range(nr)]
            for d in descs:
                d.wait()
            for j in range(nr):
                pltpu.sync_copy(rows.at[pl.ds(j * BATCH, BATCH)],
                                accg.at[didx.at[j, 0]], add=True)
                if with_ea:
                    pltpu.sync_copy(earows.at[pl.ds(j * BATCH, BATCH)],
                                    acce.at[didx.at[j, 0]], add=True)

        base = wid * rows_per

        def slab_loop(t, carry):
            do_slab(base + t * SLAB, SLAB)
            return carry
        lax.fori_loop(0, n_slabs, slab_loop, 0)
        if rem_rows:
            do_slab(base + n_slabs * SLAB, rem_rows)
        if tail:
            @pl.when(wid < tail)
            def _():
                do_slab(nw * rows_per + wid, 1)

        plsc.subcore_barrier()
        pltpu.sync_copy(accg.at[sl], outg.at[c, sl])
        if with_ea:
            pltpu.sync_copy(acce.at[sl], oute.at[c, sl])

    return pl.kernel(body, out_type=tuple(out_types), mesh=mesh,
                     scratch_types=scratch,
                     compiler_params=pltpu.CompilerParams(
                         use_tc_tiling_on_sc=False))


def kernel(x, edge_index, edge_attr, W1n, W1e, b1, Ws1, bs1,
           W2n, W2e, b2, Ws2, bs2, W3, b3):
    N, D = x.shape
    E = edge_index.shape[1]
    DE = edge_attr.shape[1]
    H = W1n.shape[1]
    R = E // BATCH
    src1 = edge_index[0]
    dst1 = edge_index[1]

    wa = jnp.concatenate([W1n, Ws1], axis=1)

    def pre_body(x_ref, w_ref, o_ref):
        o_ref[...] = jnp.dot(x_ref[...], w_ref[...], preferred_element_type=F32)

    a = pl.pallas_call(
        pre_body, out_shape=jax.ShapeDtypeStruct((N, 2 * H), F32))(x, wa)
    p1 = a[:, :H]

    g1p, eap = _seg_sum(N, R, H, True)(p1, src1, dst1, edge_attr)

    def mid_body(g1_ref, ea_ref, a_ref, w1e_ref, w2e_ref, w2n_ref, ws2_ref,
                 b1_ref, bs1_ref, b2_ref, bs2_ref, p2_ref, t_ref):
        ea = ea_ref[0, :N] + ea_ref[1, :N]
        agg1 = (g1_ref[0, :N] + g1_ref[1, :N]
                + jnp.dot(ea, w1e_ref[...], preferred_element_type=F32)
                + b1_ref[...])
        hh = jnp.maximum(agg1 + a_ref[:, H:] + bs1_ref[...], 0.0)
        p2_ref[...] = jnp.dot(hh, w2n_ref[...], preferred_element_type=F32)
        t_ref[...] = (jnp.dot(ea, w2e_ref[...], preferred_element_type=F32)
                      + b2_ref[...]
                      + jnp.dot(hh, ws2_ref[...], preferred_element_type=F32)
                      + bs2_ref[...])

    p2, t = pl.pallas_call(
        mid_body,
        out_shape=[jax.ShapeDtypeStruct((N, H), F32)] * 2,
    )(g1p, eap, a, W1e, W2e, W2n, Ws2,
      b1.reshape(1, H), bs1.reshape(1, H), b2.reshape(1, H), bs2.reshape(1, H))

    (g2p,) = _seg_sum(N, R, H, False)(p2, src1, dst1)

    def out_body(g2_ref, t_ref, w3_ref, b3_ref, o_ref):
        h2 = g2_ref[0, :N] + g2_ref[1, :N] + t_ref[...]
        o_ref[...] = (jnp.dot(h2, w3_ref[...], preferred_element_type=F32)
                      + b3_ref[...])

    return pl.pallas_call(
        out_body, out_shape=jax.ShapeDtypeStruct((N, D), F32))(
            g2p, t, W3, b3.reshape(1, D))


# speedup vs baseline: 1.6554x; 1.6554x over previous
import jax
import jax.numpy as jnp
from jax import lax
from jax.experimental import pallas as pl
from jax.experimental.pallas import tpu as pltpu
from jax.experimental.pallas import tpu_sc as plsc

F32 = jnp.float32
NC, NS = 2, 16
BATCH = 128
SLAB = 13


def _plan(n_rows):
    nw = NC * NS
    rows_per = n_rows // nw
    tail = n_rows - rows_per * nw
    slabs = []
    r = 0
    while r < rows_per:
        nr = min(SLAB, rows_per - r)
        slabs.append((r, nr))
        r += nr
    return rows_per, tail, slabs


def _acc_geometry(n_nodes):
    per_sub = -(-n_nodes // NS)
    per_sub += (-per_sub) % 8
    return per_sub, per_sub * NS


def _zero_and_barrier(zbuf, accs, s, per_sub, h):
    def zloop(i, carry):
        zbuf[i] = jnp.zeros((h,), F32)
        return carry
    lax.fori_loop(0, per_sub, zloop, 0)
    sl = pl.ds(s * per_sub, per_sub)
    for acc in accs:
        pltpu.sync_copy(zbuf, acc.at[sl])
    plsc.subcore_barrier()
    return sl


def _sc_gather_segsum(n_nodes, n_rows, h):
    rows_per, tail, slabs = _plan(n_rows)
    per_sub, n_pad = _acc_geometry(n_nodes)
    mesh = plsc.VectorSubcoreMesh(core_axis_name="c", subcore_axis_name="s")
    scratch = [
        pltpu.VMEM((per_sub, h), F32),
        pltpu.VMEM((SLAB, 1, BATCH), jnp.int32),
        pltpu.VMEM((SLAB, 1, BATCH), jnp.int32),
        pltpu.VMEM((SLAB, 1, BATCH), jnp.int32),
        pltpu.VMEM((SLAB * BATCH, h), F32),
        pltpu.VMEM((SLAB * BATCH, h), F32),
        pltpu.SemaphoreType.DMA,
        pltpu.SemaphoreType.DMA,
        pltpu.VMEM_SHARED((n_pad, h), F32),
    ]

    def body(table, src2, dst2, outg, zbuf, didxa, didxb, sidx,
             rowsa, rowsb, semg, sems, accg):
        c = lax.axis_index("c")
        s = lax.axis_index("s")
        wid = c * NS + s
        sl = _zero_and_barrier(zbuf, [accg], s, per_sub, h)
        base = wid * rows_per

        pend = {}
        for ti, (r0, nr) in enumerate(slabs):
            buf = rowsa if ti % 2 == 0 else rowsb
            dbuf = didxa if ti % 2 == 0 else didxb
            if ti >= 2:
                for d in pend.pop(ti - 2):
                    d.wait()
            pltpu.sync_copy(dst2.at[pl.ds(base + r0, nr)],
                            dbuf.at[pl.ds(0, nr)])
            pltpu.sync_copy(src2.at[pl.ds(base + r0, nr)],
                            sidx.at[pl.ds(0, nr)])
            gd = [pltpu.async_copy(table.at[sidx.at[j, 0]],
                                   buf.at[pl.ds(j * BATCH, BATCH)], semg)
                  for j in range(nr)]
            for d in gd:
                d.wait()
            pend[ti] = [pltpu.async_copy(buf.at[pl.ds(j * BATCH, BATCH)],
                                         accg.at[dbuf.at[j, 0]], sems,
                                         add=True)
                        for j in range(nr)]
        for ds in pend.values():
            for d in ds:
                d.wait()
        if tail:
            @pl.when(wid < tail)
            def _():
                r = NC * NS * rows_per + wid
                pltpu.sync_copy(dst2.at[pl.ds(r, 1)], didxa.at[pl.ds(0, 1)])
                pltpu.sync_copy(src2.at[pl.ds(r, 1)], sidx.at[pl.ds(0, 1)])
                pltpu.async_copy(table.at[sidx.at[0, 0]],
                                 rowsa.at[pl.ds(0, BATCH)], semg).wait()
                pltpu.sync_copy(rowsa.at[pl.ds(0, BATCH)],
                                accg.at[didxa.at[0, 0]], add=True)

        plsc.subcore_barrier()
        pltpu.sync_copy(accg.at[sl], outg.at[c, sl])

    return pl.kernel(
        body,
        out_type=jax.ShapeDtypeStruct((NC, n_pad, h), F32),
        mesh=mesh, scratch_types=scratch,
        compiler_params=pltpu.CompilerParams(use_tc_tiling_on_sc=False))


def _sc_ea_segsum(n_nodes, n_rows, h):
    rows_per, tail, slabs = _plan(n_rows)
    per_sub, n_pad = _acc_geometry(n_nodes)
    mesh = plsc.VectorSubcoreMesh(core_axis_name="c", subcore_axis_name="s")
    scratch = [
        pltpu.VMEM((per_sub, h), F32),
        pltpu.VMEM((SLAB, 1, BATCH), jnp.int32),
        pltpu.VMEM((SLAB, 1, BATCH), jnp.int32),
        pltpu.VMEM((SLAB, BATCH, h), F32),
        pltpu.VMEM((SLAB, BATCH, h), F32),
        pltpu.SemaphoreType.DMA,
        pltpu.SemaphoreType.DMA,
        pltpu.VMEM_SHARED((n_pad, h), F32),
    ]

    def body(ea3, dst2, oute, zbuf, didxa, didxb, ebufa, ebufb,
             semg, sems, acce):
        c = lax.axis_index("c")
        s = lax.axis_index("s")
        wid = c * NS + s
        sl = _zero_and_barrier(zbuf, [acce], s, per_sub, h)
        base = wid * rows_per

        pend = {}
        for ti, (r0, nr) in enumerate(slabs):
            buf = ebufa if ti % 2 == 0 else ebufb
            dbuf = didxa if ti % 2 == 0 else didxb
            if ti >= 2:
                for d in pend.pop(ti - 2):
                    d.wait()
            pltpu.sync_copy(dst2.at[pl.ds(base + r0, nr)],
                            dbuf.at[pl.ds(0, nr)])
            pltpu.async_copy(ea3.at[pl.ds(base + r0, nr)],
                             buf.at[pl.ds(0, nr)], semg).wait()
            pend[ti] = [pltpu.async_copy(buf.at[j],
                                         acce.at[dbuf.at[j, 0]], sems,
                                         add=True)
                        for j in range(nr)]
        for ds in pend.values():
            for d in ds:
                d.wait()
        if tail:
            @pl.when(wid < tail)
            def _():
                r = NC * NS * rows_per + wid
                pltpu.sync_copy(dst2.at[pl.ds(r, 1)], didxa.at[pl.ds(0, 1)])
                pltpu.sync_copy(ea3.at[pl.ds(r, 1)], ebufa.at[pl.ds(0, 1)])
                pltpu.sync_copy(ebufa.at[0], acce.at[didxa.at[0, 0]],
                                add=True)

        plsc.subcore_barrier()
        pltpu.sync_copy(acce.at[sl], oute.at[c, sl])

    return pl.kernel(
        body,
        out_type=jax.ShapeDtypeStruct((NC, n_pad, h), F32),
        mesh=mesh, scratch_types=scratch,
        compiler_params=pltpu.CompilerParams(use_tc_tiling_on_sc=False))


def kernel(x, edge_index, edge_attr, W1n, W1e, b1, Ws1, bs1,
           W2n, W2e, b2, Ws2, bs2, W3, b3):
    N, D = x.shape
    E = edge_index.shape[1]
    DE = edge_attr.shape[1]
    H = W1n.shape[1]
    R = E // BATCH
    src2 = edge_index[0].reshape(R, 1, BATCH)
    dst2 = edge_index[1].reshape(R, 1, BATCH)
    ea3 = edge_attr.reshape(R, BATCH, DE)

    def pre_body(x_ref, w1_ref, ws_ref, p1_ref, s1_ref):
        xv = x_ref[...]
        p1_ref[...] = jnp.dot(xv, w1_ref[...], preferred_element_type=F32)
        s1_ref[...] = jnp.dot(xv, ws_ref[...], preferred_element_type=F32)

    p1, s1 = pl.pallas_call(
        pre_body,
        out_shape=[jax.ShapeDtypeStruct((N, H), F32)] * 2)(x, W1n, Ws1)

    g1p = _sc_gather_segsum(N, R, H)(p1, src2, dst2)
    eap = _sc_ea_segsum(N, R, DE)(ea3, dst2)

    def mid_body(g1_ref, ea_ref, s1_ref, w1e_ref, w2e_ref, w2n_ref, ws2_ref,
                 b1_ref, bs1_ref, b2_ref, bs2_ref, p2_ref, t_ref):
        ea = ea_ref[0, :N] + ea_ref[1, :N]
        agg1 = (g1_ref[0, :N] + g1_ref[1, :N]
                + jnp.dot(ea, w1e_ref[...], preferred_element_type=F32)
                + b1_ref[...])
        hh = jnp.maximum(agg1 + s1_ref[...] + bs1_ref[...], 0.0)
        p2_ref[...] = jnp.dot(hh, w2n_ref[...], preferred_element_type=F32)
        t_ref[...] = (jnp.dot(ea, w2e_ref[...], preferred_element_type=F32)
                      + b2_ref[...]
                      + jnp.dot(hh, ws2_ref[...], preferred_element_type=F32)
                      + bs2_ref[...])

    p2, t = pl.pallas_call(
        mid_body,
        out_shape=[jax.ShapeDtypeStruct((N, H), F32)] * 2,
    )(g1p, eap, s1, W1e, W2e, W2n, Ws2,
      b1.reshape(1, H), bs1.reshape(1, H), b2.reshape(1, H), bs2.reshape(1, H))

    g2p = _sc_gather_segsum(N, R, H)(p2, src2, dst2)

    def out_body(g2_ref, t_ref, w3_ref, b3_ref, o_ref):
        h2 = g2_ref[0, :N] + g2_ref[1, :N] + t_ref[...]
        o_ref[...] = (jnp.dot(h2, w3_ref[...], preferred_element_type=F32)
                      + b3_ref[...])

    return pl.pallas_call(
        out_body, out_shape=jax.ShapeDtypeStruct((N, D), F32))(
            g2p, t, W3, b3.reshape(1, D))
